# baseline (speedup 1.0000x reference)
import functools

import jax
import jax.numpy as jnp
from jax import lax
from jax.experimental import pallas as pl
from jax.experimental.pallas import tpu as pltpu
from jax.experimental.pallas import tpu_sc as plsc


def _argmin_body(x_ref, c_ref, out_ref, *, tm, tk, nk):
    x = x_ref[...]
    x_sq = jnp.sum(x * x, axis=1, keepdims=True)

    def step(j, carry):
        pmin, parg = carry
        c = c_ref[pl.ds(j * tk, tk), :]
        c_sq = jnp.sum(c * c, axis=1)
        cross = lax.dot_general(x, c, (((1,), (1,)), ((), ())),
                                preferred_element_type=jnp.float32)
        d2 = x_sq + c_sq[None, :] - 2.0 * cross
        dist = jnp.sqrt(jnp.maximum(d2, 0.0))
        lmin = jnp.min(dist, axis=1, keepdims=True)
        idx = lax.broadcasted_iota(jnp.int32, (tm, tk), 1)
        larg = jnp.min(jnp.where(dist == lmin, idx, jnp.int32(nk)),
                       axis=1, keepdims=True) + j * tk
        better = lmin < pmin
        return jnp.where(better, lmin, pmin), jnp.where(better, larg, parg)

    init = (jnp.full((tm, 1), jnp.inf, jnp.float32),
            jnp.zeros((tm, 1), jnp.int32))
    _, arg = lax.fori_loop(0, nk // tk, step, init)
    out_ref[...] = arg


def _nearest_cluster(xf, centers, *, tm, tk):
    m, d = xf.shape
    k = centers.shape[0]
    sel = pl.pallas_call(
        functools.partial(_argmin_body, tm=tm, tk=tk, nk=k),
        grid=(m // tm,),
        in_specs=[
            pl.BlockSpec((tm, d), lambda i: (i, 0)),
            pl.BlockSpec((k, d), lambda i: (0, 0)),
        ],
        out_specs=pl.BlockSpec((tm, 1), lambda i: (i, 0)),
        out_shape=jax.ShapeDtypeStruct((m, 1), jnp.int32),
    )(xf, centers)
    return sel.reshape(m)


def _make_sc_gather(v, d, b):
    info = plsc.get_sparse_core_info()
    nw = info.num_cores * info.num_subcores
    assert b % (8 * nw) == 0
    b_per_w = b // nw
    mesh = plsc.VectorSubcoreMesh(core_axis_name="c", subcore_axis_name="s")

    @functools.partial(
        pl.kernel, mesh=mesh,
        out_type=jax.ShapeDtypeStruct((b, d), jnp.float32),
        scratch_types=[
            pltpu.VMEM((b_per_w,), jnp.int32),
            pltpu.VMEM((b_per_w, d), jnp.float32),
            pltpu.SemaphoreType.DMA,
        ],
    )
    def gather(table_hbm, idx_hbm, out_hbm, idx_v, rows_v, sem):
        wid = lax.axis_index("s") * info.num_cores + lax.axis_index("c")
        base = wid * b_per_w
        pltpu.sync_copy(idx_hbm.at[pl.ds(base, b_per_w)], idx_v)
        pltpu.async_copy(table_hbm.at[idx_v], rows_v, sem).wait()
        pltpu.sync_copy(rows_v, out_hbm.at[pl.ds(base, b_per_w)])

    return gather


def _proj_body(q_ref, w_ref, b_ref, out_ref):
    out_ref[...] = lax.dot_general(
        q_ref[...], w_ref[...], (((1,), (1,)), ((), ())),
        preferred_element_type=jnp.float32) + b_ref[...]


def _project(q, w, bias, *, tp):
    m, d = q.shape
    o = w.shape[0]
    return pl.pallas_call(
        _proj_body,
        grid=(m // tp,),
        in_specs=[
            pl.BlockSpec((tp, d), lambda i: (i, 0)),
            pl.BlockSpec((o, d), lambda i: (0, 0)),
            pl.BlockSpec((1, o), lambda i: (0, 0)),
        ],
        out_specs=pl.BlockSpec((tp, o), lambda i: (i, 0)),
        out_shape=jax.ShapeDtypeStruct((m, o), jnp.float32),
    )(q, w, bias.reshape(1, o))


def kernel(x, cluster_centers, W, b):
    bb, s, d = x.shape
    m = bb * s
    xf = x.reshape(m, d)
    selected = _nearest_cluster(xf, cluster_centers, tm=256, tk=2048)
    q = _make_sc_gather(cluster_centers.shape[0], d, m)(cluster_centers, selected)
    out = _project(q, W, b, tp=512)
    return out.reshape(bb, s, W.shape[0])


# speedup vs baseline: 1.5541x; 1.5541x over previous
import functools

import jax
import jax.numpy as jnp
from jax import lax
from jax.experimental import pallas as pl
from jax.experimental.pallas import tpu as pltpu
from jax.experimental.pallas import tpu_sc as plsc


def _csq_body(c_ref, out_ref):
    c = c_ref[...]
    out_ref[...] = jnp.sum(c * c, axis=1, keepdims=True)


def _cluster_sq(centers):
    k, d = centers.shape
    csq = pl.pallas_call(
        _csq_body,
        out_shape=jax.ShapeDtypeStruct((k, 1), jnp.float32),
    )(centers)
    return csq.reshape(1, k)


def _argmin_body(x_ref, c_ref, csq_ref, out_ref, d2_ref, *, tm, tk, nk):
    x = x_ref[...]
    x_sq = jnp.sum(x * x, axis=1, keepdims=True)
    nj = nk // tk

    m = jnp.full((tm, 1), jnp.inf, jnp.float32)
    for j in range(nj):
        c = c_ref[pl.ds(j * tk, tk), :]
        c_sq = csq_ref[:, pl.ds(j * tk, tk)]
        cross = lax.dot_general(x, c, (((1,), (1,)), ((), ())),
                                preferred_element_type=jnp.float32)
        d2 = (x_sq + c_sq) - 2.0 * cross
        d2_ref[:, pl.ds(j * tk, tk)] = d2
        m = jnp.minimum(m, jnp.min(d2, axis=1, keepdims=True))

    s = jnp.sqrt(m)
    mb = lax.bitcast_convert_type(m, jnp.int32)
    hi = m
    for u in range(1, 6):
        t = lax.bitcast_convert_type(mb + u, jnp.float32)
        hi = jnp.where(jnp.sqrt(t) == s, t, hi)

    idx = lax.broadcasted_iota(jnp.int32, (tm, tk), 1)
    arg = jnp.full((tm, 1), jnp.int32(1 << 30), jnp.int32)
    for j in range(nj):
        d2 = d2_ref[:, pl.ds(j * tk, tk)]
        larg = jnp.min(jnp.where(d2 <= hi, idx, jnp.int32(nk)),
                       axis=1, keepdims=True) + j * tk
        arg = jnp.minimum(arg, larg)
    out_ref[...] = arg


def _nearest_cluster(xf, centers, csq, *, tm, tk):
    m, d = xf.shape
    k = centers.shape[0]
    sel = pl.pallas_call(
        functools.partial(_argmin_body, tm=tm, tk=tk, nk=k),
        grid=(m // tm,),
        in_specs=[
            pl.BlockSpec((tm, d), lambda i: (i, 0)),
            pl.BlockSpec((k, d), lambda i: (0, 0)),
            pl.BlockSpec((1, k), lambda i: (0, 0)),
        ],
        out_specs=pl.BlockSpec((tm, 1), lambda i: (i, 0)),
        out_shape=jax.ShapeDtypeStruct((m, 1), jnp.int32),
        scratch_shapes=[pltpu.VMEM((tm, k), jnp.float32)],
    )(xf, centers, csq)
    return sel.reshape(m)


def _make_sc_gather(v, d, b):
    info = plsc.get_sparse_core_info()
    nw = info.num_cores * info.num_subcores
    assert b % (8 * nw) == 0
    b_per_w = b // nw
    mesh = plsc.VectorSubcoreMesh(core_axis_name="c", subcore_axis_name="s")

    @functools.partial(
        pl.kernel, mesh=mesh,
        out_type=jax.ShapeDtypeStruct((b, d), jnp.float32),
        scratch_types=[
            pltpu.VMEM((b_per_w,), jnp.int32),
            pltpu.VMEM((b_per_w, d), jnp.float32),
            pltpu.SemaphoreType.DMA,
        ],
    )
    def gather(table_hbm, idx_hbm, out_hbm, idx_v, rows_v, sem):
        wid = lax.axis_index("s") * info.num_cores + lax.axis_index("c")
        base = wid * b_per_w
        pltpu.sync_copy(idx_hbm.at[pl.ds(base, b_per_w)], idx_v)
        pltpu.async_copy(table_hbm.at[idx_v], rows_v, sem).wait()
        pltpu.sync_copy(rows_v, out_hbm.at[pl.ds(base, b_per_w)])

    return gather


def _proj_body(q_ref, w_ref, b_ref, out_ref):
    out_ref[...] = lax.dot_general(
        q_ref[...], w_ref[...], (((1,), (1,)), ((), ())),
        preferred_element_type=jnp.float32) + b_ref[...]


def _project(q, w, bias, *, tp):
    m, d = q.shape
    o = w.shape[0]
    return pl.pallas_call(
        _proj_body,
        grid=(m // tp,),
        in_specs=[
            pl.BlockSpec((tp, d), lambda i: (i, 0)),
            pl.BlockSpec((o, d), lambda i: (0, 0)),
            pl.BlockSpec((1, o), lambda i: (0, 0)),
        ],
        out_specs=pl.BlockSpec((tp, o), lambda i: (i, 0)),
        out_shape=jax.ShapeDtypeStruct((m, o), jnp.float32),
    )(q, w, bias.reshape(1, o))


def kernel(x, cluster_centers, W, b):
    bb, s, d = x.shape
    m = bb * s
    xf = x.reshape(m, d)
    csq = _cluster_sq(cluster_centers)
    selected = _nearest_cluster(xf, cluster_centers, csq, tm=256, tk=2048)
    q = _make_sc_gather(cluster_centers.shape[0], d, m)(cluster_centers, selected)
    out = _project(q, W, b, tp=512)
    return out.reshape(bb, s, W.shape[0])


# speedup vs baseline: 1.6938x; 1.0899x over previous
import functools

import jax
import jax.numpy as jnp
from jax import lax
from jax.experimental import pallas as pl
from jax.experimental.pallas import tpu as pltpu
from jax.experimental.pallas import tpu_sc as plsc


def _csq_body(c_ref, out_ref):
    c = c_ref[...]
    out_ref[...] = jnp.sum(c * c, axis=1, keepdims=True)


def _cluster_sq(centers):
    k, d = centers.shape
    csq = pl.pallas_call(
        _csq_body,
        out_shape=jax.ShapeDtypeStruct((k, 1), jnp.float32),
    )(centers)
    return csq.reshape(1, k)


def _argmin_body(x_ref, c_ref, csq_ref, out_ref, *, tm, tk, nk):
    x = x_ref[...]
    x_sq = jnp.sum(x * x, axis=1, keepdims=True)
    nj = nk // tk
    nc = tk // 128

    v = jnp.full((tm, 128), jnp.inf, jnp.float32)
    a = jnp.zeros((tm, 128), jnp.int32)
    for j in range(nj):
        c = c_ref[pl.ds(j * tk, tk), :]
        c_sq = csq_ref[:, pl.ds(j * tk, tk)]
        cross = lax.dot_general(x, c, (((1,), (1,)), ((), ())),
                                preferred_element_type=jnp.float32)
        d2 = (x_sq + c_sq) - 2.0 * cross
        for ch in range(nc):
            blk = d2[:, ch * 128:(ch + 1) * 128]
            upd = blk < v
            v = jnp.where(upd, blk, v)
            a = jnp.where(upd, jnp.int32(j * nc + ch), a)

    m = jnp.min(v, axis=1, keepdims=True)
    s = jnp.sqrt(m)
    mb = lax.bitcast_convert_type(m, jnp.int32)
    hi = m
    for u in range(1, 6):
        t = lax.bitcast_convert_type(mb + u, jnp.float32)
        hi = jnp.where(jnp.sqrt(t) == s, t, hi)

    lane = lax.broadcasted_iota(jnp.int32, (tm, 128), 1)
    g = a * 128 + lane
    arg = jnp.min(jnp.where(v <= hi, g, jnp.int32(1 << 30)),
                  axis=1, keepdims=True)
    out_ref[...] = arg


def _nearest_cluster(xf, centers, csq, *, tm, tk):
    m, d = xf.shape
    k = centers.shape[0]
    sel = pl.pallas_call(
        functools.partial(_argmin_body, tm=tm, tk=tk, nk=k),
        grid=(m // tm,),
        in_specs=[
            pl.BlockSpec((tm, d), lambda i: (i, 0)),
            pl.BlockSpec((k, d), lambda i: (0, 0)),
            pl.BlockSpec((1, k), lambda i: (0, 0)),
        ],
        out_specs=pl.BlockSpec((tm, 1), lambda i: (i, 0)),
        out_shape=jax.ShapeDtypeStruct((m, 1), jnp.int32),
    )(xf, centers, csq)
    return sel.reshape(m)


def _make_sc_gather(v, d, b):
    info = plsc.get_sparse_core_info()
    nw = info.num_cores * info.num_subcores
    assert b % (8 * nw) == 0
    b_per_w = b // nw
    mesh = plsc.VectorSubcoreMesh(core_axis_name="c", subcore_axis_name="s")

    @functools.partial(
        pl.kernel, mesh=mesh,
        out_type=jax.ShapeDtypeStruct((b, d), jnp.float32),
        scratch_types=[
            pltpu.VMEM((b_per_w,), jnp.int32),
            pltpu.VMEM((b_per_w, d), jnp.float32),
            pltpu.SemaphoreType.DMA,
        ],
    )
    def gather(table_hbm, idx_hbm, out_hbm, idx_v, rows_v, sem):
        wid = lax.axis_index("s") * info.num_cores + lax.axis_index("c")
        base = wid * b_per_w
        pltpu.sync_copy(idx_hbm.at[pl.ds(base, b_per_w)], idx_v)
        pltpu.async_copy(table_hbm.at[idx_v], rows_v, sem).wait()
        pltpu.sync_copy(rows_v, out_hbm.at[pl.ds(base, b_per_w)])

    return gather


def _proj_body(q_ref, w_ref, b_ref, out_ref):
    out_ref[...] = lax.dot_general(
        q_ref[...], w_ref[...], (((1,), (1,)), ((), ())),
        preferred_element_type=jnp.float32) + b_ref[...]


def _project(q, w, bias, *, tp):
    m, d = q.shape
    o = w.shape[0]
    return pl.pallas_call(
        _proj_body,
        grid=(m // tp,),
        in_specs=[
            pl.BlockSpec((tp, d), lambda i: (i, 0)),
            pl.BlockSpec((o, d), lambda i: (0, 0)),
            pl.BlockSpec((1, o), lambda i: (0, 0)),
        ],
        out_specs=pl.BlockSpec((tp, o), lambda i: (i, 0)),
        out_shape=jax.ShapeDtypeStruct((m, o), jnp.float32),
    )(q, w, bias.reshape(1, o))


def kernel(x, cluster_centers, W, b):
    bb, s, d = x.shape
    m = bb * s
    xf = x.reshape(m, d)
    csq = _cluster_sq(cluster_centers)
    selected = _nearest_cluster(xf, cluster_centers, csq, tm=256, tk=2048)
    q = _make_sc_gather(cluster_centers.shape[0], d, m)(cluster_centers, selected)
    out = _project(q, W, b, tp=512)
    return out.reshape(bb, s, W.shape[0])


# speedup vs baseline: 1.8135x; 1.0707x over previous
import functools

import jax
import jax.numpy as jnp
from jax import lax
from jax.experimental import pallas as pl
from jax.experimental.pallas import tpu as pltpu
from jax.experimental.pallas import tpu_sc as plsc


def _argmin_body(x_ref, c_ref, out_ref, csq_ref, *, tm, tk, nk):
    nj = nk // tk
    nc = tk // 128

    @pl.when(pl.program_id(0) == 0)
    def _():
        for j in range(nj):
            cj = c_ref[pl.ds(j * tk, tk), :]
            csq_ref[:, pl.ds(j * tk, tk)] = jnp.sum(cj * cj, axis=1)[None, :]

    x = x_ref[...]
    x_sq = jnp.sum(x * x, axis=1, keepdims=True)

    v = jnp.full((tm, 128), jnp.inf, jnp.float32)
    a = jnp.zeros((tm, 128), jnp.int32)
    for j in range(nj):
        c = c_ref[pl.ds(j * tk, tk), :]
        c_sq = csq_ref[:, pl.ds(j * tk, tk)]
        cross = lax.dot_general(x, c, (((1,), (1,)), ((), ())),
                                preferred_element_type=jnp.float32)
        d2 = (x_sq + c_sq) - 2.0 * cross
        for ch in range(nc):
            blk = d2[:, ch * 128:(ch + 1) * 128]
            upd = blk < v
            v = jnp.where(upd, blk, v)
            a = jnp.where(upd, jnp.int32(j * nc + ch), a)

    m = jnp.min(v, axis=1, keepdims=True)
    s = jnp.sqrt(m)
    mb = lax.bitcast_convert_type(m, jnp.int32)
    hi = m
    for u in range(1, 6):
        t = lax.bitcast_convert_type(mb + u, jnp.float32)
        hi = jnp.where(jnp.sqrt(t) == s, t, hi)

    lane = lax.broadcasted_iota(jnp.int32, (tm, 128), 1)
    g = a * 128 + lane
    arg = jnp.min(jnp.where(v <= hi, g, jnp.int32(1 << 30)),
                  axis=1, keepdims=True)
    out_ref[...] = arg


def _nearest_cluster(xf, centers, *, tm, tk):
    m, d = xf.shape
    k = centers.shape[0]
    sel = pl.pallas_call(
        functools.partial(_argmin_body, tm=tm, tk=tk, nk=k),
        grid=(m // tm,),
        in_specs=[
            pl.BlockSpec((tm, d), lambda i: (i, 0)),
            pl.BlockSpec((k, d), lambda i: (0, 0)),
        ],
        out_specs=pl.BlockSpec((tm, 1), lambda i: (i, 0)),
        out_shape=jax.ShapeDtypeStruct((m, 1), jnp.int32),
        scratch_shapes=[pltpu.VMEM((1, k), jnp.float32)],
    )(xf, centers)
    return sel.reshape(m)


def _make_sc_gather(v, d, b):
    info = plsc.get_sparse_core_info()
    nw = info.num_cores * info.num_subcores
    assert b % (8 * nw) == 0
    b_per_w = b // nw
    mesh = plsc.VectorSubcoreMesh(core_axis_name="c", subcore_axis_name="s")

    @functools.partial(
        pl.kernel, mesh=mesh,
        out_type=jax.ShapeDtypeStruct((b, d), jnp.float32),
        scratch_types=[
            pltpu.VMEM((b_per_w,), jnp.int32),
            pltpu.VMEM((b_per_w, d), jnp.float32),
            pltpu.SemaphoreType.DMA,
        ],
    )
    def gather(table_hbm, idx_hbm, out_hbm, idx_v, rows_v, sem):
        wid = lax.axis_index("s") * info.num_cores + lax.axis_index("c")
        base = wid * b_per_w
        pltpu.sync_copy(idx_hbm.at[pl.ds(base, b_per_w)], idx_v)
        pltpu.async_copy(table_hbm.at[idx_v], rows_v, sem).wait()
        pltpu.sync_copy(rows_v, out_hbm.at[pl.ds(base, b_per_w)])

    return gather


def _proj_body(q_ref, w_ref, b_ref, out_ref):
    out_ref[...] = lax.dot_general(
        q_ref[...], w_ref[...], (((1,), (1,)), ((), ())),
        preferred_element_type=jnp.float32) + b_ref[...]


def _project(q, w, bias, *, tp):
    m, d = q.shape
    o = w.shape[0]
    return pl.pallas_call(
        _proj_body,
        grid=(m // tp,),
        in_specs=[
            pl.BlockSpec((tp, d), lambda i: (i, 0)),
            pl.BlockSpec((o, d), lambda i: (0, 0)),
            pl.BlockSpec((1, o), lambda i: (0, 0)),
        ],
        out_specs=pl.BlockSpec((tp, o), lambda i: (i, 0)),
        out_shape=jax.ShapeDtypeStruct((m, o), jnp.float32),
    )(q, w, bias.reshape(1, o))


def kernel(x, cluster_centers, W, b):
    bb, s, d = x.shape
    m = bb * s
    xf = x.reshape(m, d)
    selected = _nearest_cluster(xf, cluster_centers, tm=256, tk=2048)
    q = _make_sc_gather(cluster_centers.shape[0], d, m)(cluster_centers, selected)
    out = _project(q, W, b, tp=512)
    return out.reshape(bb, s, W.shape[0])


# speedup vs baseline: 1.9051x; 1.0505x over previous
import functools

import jax
import jax.numpy as jnp
from jax import lax
from jax.experimental import pallas as pl
from jax.experimental.pallas import tpu as pltpu
from jax.experimental.pallas import tpu_sc as plsc


def _argmin_body(x_ref, c_ref, w_ref, b_ref, out_ref, cproj_ref, csq_ref,
                 *, tm, tk, nk, tp):
    i = pl.program_id(0)
    nj = nk // tk
    nc = tk // 128
    npj = nk // tp

    @pl.when(i == 0)
    def _():
        for j in range(nj):
            cj = c_ref[pl.ds(j * tk, tk), :]
            csq_ref[:, pl.ds(j * tk, tk)] = jnp.sum(cj * cj, axis=1)[None, :]

    @pl.when(i < npj)
    def _():
        base = jnp.minimum(i, npj - 1) * tp
        cp = c_ref[pl.ds(base, tp), :]
        cproj_ref[...] = lax.dot_general(
            cp, w_ref[...], (((1,), (1,)), ((), ())),
            preferred_element_type=jnp.float32) + b_ref[...]

    x = x_ref[...]
    x_sq = jnp.sum(x * x, axis=1, keepdims=True)

    v = jnp.full((tm, 128), jnp.inf, jnp.float32)
    a = jnp.zeros((tm, 128), jnp.int32)
    for j in range(nj):
        c = c_ref[pl.ds(j * tk, tk), :]
        c_sq = csq_ref[:, pl.ds(j * tk, tk)]
        cross = lax.dot_general(x, c, (((1,), (1,)), ((), ())),
                                preferred_element_type=jnp.float32)
        d2 = (x_sq + c_sq) - 2.0 * cross
        for ch in range(nc):
            blk = d2[:, ch * 128:(ch + 1) * 128]
            upd = blk < v
            v = jnp.where(upd, blk, v)
            a = jnp.where(upd, jnp.int32(j * nc + ch), a)

    m = jnp.min(v, axis=1, keepdims=True)
    s = jnp.sqrt(m)
    mb = lax.bitcast_convert_type(m, jnp.int32)
    hi = m
    for u in range(1, 6):
        t = lax.bitcast_convert_type(mb + u, jnp.float32)
        hi = jnp.where(jnp.sqrt(t) == s, t, hi)

    lane = lax.broadcasted_iota(jnp.int32, (tm, 128), 1)
    g = a * 128 + lane
    arg = jnp.min(jnp.where(v <= hi, g, jnp.int32(1 << 30)),
                  axis=1, keepdims=True)
    out_ref[...] = arg


def _nearest_cluster_and_proj(xf, centers, w, bias, *, tm, tk):
    m, d = xf.shape
    k = centers.shape[0]
    ni = m // tm
    tp = k // (ni - 2)
    sel, cproj = pl.pallas_call(
        functools.partial(_argmin_body, tm=tm, tk=tk, nk=k, tp=tp),
        grid=(ni,),
        in_specs=[
            pl.BlockSpec((tm, d), lambda i: (i, 0)),
            pl.BlockSpec((k, d), lambda i: (0, 0)),
            pl.BlockSpec((d, d), lambda i: (0, 0)),
            pl.BlockSpec((1, d), lambda i: (0, 0)),
        ],
        out_specs=[
            pl.BlockSpec((tm, 1), lambda i: (i, 0)),
            pl.BlockSpec((tp, d), lambda i: (jnp.minimum(i, k // tp - 1), 0)),
        ],
        out_shape=[
            jax.ShapeDtypeStruct((m, 1), jnp.int32),
            jax.ShapeDtypeStruct((k, d), jnp.float32),
        ],
        scratch_shapes=[pltpu.VMEM((1, k), jnp.float32)],
    )(xf, centers, w, bias.reshape(1, d))
    return sel.reshape(m), cproj


def _make_sc_gather(v, d, b):
    info = plsc.get_sparse_core_info()
    nw = info.num_cores * info.num_subcores
    assert b % (8 * nw) == 0
    b_per_w = b // nw
    mesh = plsc.VectorSubcoreMesh(core_axis_name="c", subcore_axis_name="s")

    @functools.partial(
        pl.kernel, mesh=mesh,
        out_type=jax.ShapeDtypeStruct((b, d), jnp.float32),
        scratch_types=[
            pltpu.VMEM((b_per_w,), jnp.int32),
            pltpu.VMEM((b_per_w, d), jnp.float32),
            pltpu.SemaphoreType.DMA,
        ],
    )
    def gather(table_hbm, idx_hbm, out_hbm, idx_v, rows_v, sem):
        wid = lax.axis_index("s") * info.num_cores + lax.axis_index("c")
        base = wid * b_per_w
        pltpu.sync_copy(idx_hbm.at[pl.ds(base, b_per_w)], idx_v)
        pltpu.async_copy(table_hbm.at[idx_v], rows_v, sem).wait()
        pltpu.sync_copy(rows_v, out_hbm.at[pl.ds(base, b_per_w)])

    return gather


def kernel(x, cluster_centers, W, b):
    bb, s, d = x.shape
    m = bb * s
    xf = x.reshape(m, d)
    selected, cproj = _nearest_cluster_and_proj(
        xf, cluster_centers, W, b, tm=256, tk=2048)
    out = _make_sc_gather(cluster_centers.shape[0], d, m)(cproj, selected)
    return out.reshape(bb, s, W.shape[0])
